# baseline (speedup 1.0000x reference)
import functools

import jax
import jax.numpy as jnp
from jax import lax
from jax.experimental import pallas as pl
from jax.experimental.pallas import tpu as pltpu
from jax.experimental.pallas import tpu_sc as plsc

VOCAB = 9048
VEC = 100
KPAD = 128
D_MODEL = 1024
B, L = 64, 512
N = B * L

NC, NS = 2, 16
NW = NC * NS
ROWS_PER_W = N // NW
CHUNK = 128
NCHUNK = ROWS_PER_W // CHUNK
NBUF = 6
AHEAD = 3


def _sc_gather(table_pad, idx3):
    mesh = plsc.VectorSubcoreMesh(core_axis_name="c", subcore_axis_name="s")

    @functools.partial(
        pl.kernel,
        mesh=mesh,
        out_type=jax.ShapeDtypeStruct((N, KPAD), jnp.float32),
        scratch_types=[
            pltpu.VMEM((NCHUNK, CHUNK), jnp.int32),
        ]
        + [pltpu.VMEM((CHUNK, KPAD), jnp.float32) for _ in range(NBUF)]
        + [pltpu.SemaphoreType.DMA for _ in range(2 * NBUF)],
    )
    def k(table_hbm, idx_hbm, out_hbm, idx_v, *scratch):
        bufs = scratch[:NBUF]
        gsems = scratch[NBUF : 2 * NBUF]
        wsems = scratch[2 * NBUF :]
        wid = lax.axis_index("s") * NC + lax.axis_index("c")
        base = wid * ROWS_PER_W
        pltpu.sync_copy(idx_hbm.at[wid], idx_v)

        gcopies = [None] * NBUF
        wcopies = [None] * NBUF

        def fire_gather(c):
            s = c % NBUF
            gcopies[s] = pltpu.async_copy(table_hbm.at[idx_v.at[c]], bufs[s], gsems[s])

        for c in range(min(AHEAD, NCHUNK)):
            fire_gather(c)
        for c in range(NCHUNK):
            s = c % NBUF
            nxt = c + AHEAD
            if nxt < NCHUNK:
                if nxt >= NBUF:
                    wcopies[nxt % NBUF].wait()
                fire_gather(nxt)
            gcopies[s].wait()
            wcopies[s] = pltpu.async_copy(
                bufs[s], out_hbm.at[pl.ds(base + c * CHUNK, CHUNK)], wsems[s]
            )
        for c in range(max(0, NCHUNK - NBUF), NCHUNK):
            wcopies[c % NBUF].wait()

    return k(table_pad, idx3)


BM = 4096


def _tc_project(x, w_pad, bias2d):
    def body(x_ref, w_ref, b_ref, o_ref):
        o_ref[...] = (
            jnp.dot(x_ref[...], w_ref[...], preferred_element_type=jnp.float32)
            + b_ref[...]
        )

    return pl.pallas_call(
        body,
        grid=(N // BM,),
        in_specs=[
            pl.BlockSpec((BM, KPAD), lambda i: (i, 0)),
            pl.BlockSpec((KPAD, D_MODEL), lambda i: (0, 0)),
            pl.BlockSpec((1, D_MODEL), lambda i: (0, 0)),
        ],
        out_specs=pl.BlockSpec((BM, D_MODEL), lambda i: (i, 0)),
        out_shape=jax.ShapeDtypeStruct((N, D_MODEL), jnp.float32),
    )(x, w_pad, bias2d)


def kernel(protX, table, W, b):
    idx3 = protX.reshape(-1).astype(jnp.int32).reshape(NW, NCHUNK, CHUNK)
    table_pad = jnp.pad(table, ((0, 0), (0, KPAD - VEC)))
    w_pad = jnp.pad(W, ((0, KPAD - VEC), (0, 0)))
    gathered = _sc_gather(table_pad, idx3)
    out = _tc_project(gathered, w_pad, b.reshape(1, D_MODEL))
    return out.reshape(B, L, D_MODEL)
